# baseline (speedup 1.0000x reference)
import jax
import jax.numpy as jnp
from jax import lax
from jax.experimental import pallas as pl
from jax.experimental.pallas import tpu as pltpu

ROWS = 128
COLS = 32768
BLK_ROWS = 16
GRID = ROWS // BLK_ROWS


def _tc_body(x_ref, out_ref):
    xb = x_ref[...]
    rowmax = jnp.max(xb, axis=1, keepdims=True)
    col = lax.broadcasted_iota(jnp.int32, xb.shape, 1)
    cand = jnp.where(xb == rowmax, col, jnp.int32(COLS))
    out_ref[...] = jnp.min(cand, axis=1).reshape(1, 1, BLK_ROWS)


@jax.jit
def _tc_argmax(x):
    return pl.pallas_call(
        _tc_body,
        grid=(GRID,),
        in_specs=[pl.BlockSpec((BLK_ROWS, COLS), lambda i: (i, 0))],
        out_specs=pl.BlockSpec((1, 1, BLK_ROWS), lambda i: (i, 0, 0)),
        out_shape=jax.ShapeDtypeStruct((GRID, 1, BLK_ROWS), jnp.int32),
    )(x)


def kernel(x):
    return _tc_argmax(x).reshape(ROWS).astype(jnp.int64)
